# baseline (speedup 1.0000x reference)
import functools

import jax
import jax.numpy as jnp
from jax import lax
from jax.experimental import pallas as pl
from jax.experimental.pallas import tpu as pltpu
from jax.experimental.pallas import tpu_sc as plsc

D = 256
LANES = 16
CHUNK = 40


def _norm_body(z_ref, o_ref):
    x = z_ref[...]
    ss = jnp.sum(x * x, axis=1, keepdims=True)
    nrm = jnp.sqrt(ss)
    o_ref[...] = x / jnp.maximum(nrm, 1e-12)


def _normalize(z):
    n, d = z.shape
    blk = 1000
    return pl.pallas_call(
        _norm_body,
        out_shape=jax.ShapeDtypeStruct((n, d), jnp.float32),
        grid=(n // blk,),
        in_specs=[pl.BlockSpec((blk, d), lambda i: (i, 0))],
        out_specs=pl.BlockSpec((blk, d), lambda i: (i, 0)),
    )(z)


@functools.cache
def _make_sc_scorer(num_edges):
    info = plsc.get_sparse_core_info()
    nc, ns = info.num_cores, info.num_subcores
    nw = nc * ns
    epw = num_edges // nw
    assert epw * nw == num_edges and epw % CHUNK == 0
    nchunks = epw // CHUNK

    mesh = plsc.VectorSubcoreMesh(core_axis_name="c", subcore_axis_name="s")

    @functools.partial(
        pl.kernel,
        mesh=mesh,
        out_type=jax.ShapeDtypeStruct((num_edges,), jnp.float32),
        compiler_params=pltpu.CompilerParams(
            use_tc_tiling_on_sc=False, needs_layout_passes=False),
        scratch_types=[
            pltpu.VMEM((epw,), jnp.int32),
            pltpu.VMEM((epw,), jnp.int32),
            pltpu.VMEM((epw,), jnp.int32),
            pltpu.VMEM((epw,), jnp.float32),
            pltpu.VMEM((CHUNK + 8, D), jnp.float32),
            pltpu.VMEM((CHUNK + 8, D), jnp.float32),
            pltpu.VMEM((CHUNK + 8, D), jnp.float32),
            pltpu.SemaphoreType.DMA,
            pltpu.SemaphoreType.DMA,
            pltpu.SemaphoreType.DMA,
        ],
    )
    def scorer(zn_hbm, src_hbm, dst_hbm, rt_hbm, rel_hbm, out_hbm,
               src_v, dst_v, rt_v, out_v, srows, drows, rrows, s1, s2, s3):
        wid = lax.axis_index("s") * nc + lax.axis_index("c")
        base = pl.multiple_of(wid * epw, 8)
        pltpu.sync_copy(src_hbm.at[pl.ds(base, epw)], src_v)
        pltpu.sync_copy(dst_hbm.at[pl.ds(base, epw)], dst_v)
        pltpu.sync_copy(rt_hbm.at[pl.ds(base, epw)], rt_v)

        lanes = lax.iota(jnp.int32, LANES)

        def chunk_body(c, carry):
            off = pl.multiple_of(c * CHUNK, 8)
            h1 = pltpu.async_copy(
                zn_hbm.at[src_v.at[pl.ds(off, CHUNK)]],
                srows.at[pl.ds(0, CHUNK)], s1)
            h2 = pltpu.async_copy(
                zn_hbm.at[dst_v.at[pl.ds(off, CHUNK)]],
                drows.at[pl.ds(0, CHUNK)], s2)
            h3 = pltpu.async_copy(
                rel_hbm.at[rt_v.at[pl.ds(off, CHUNK)]],
                rrows.at[pl.ds(0, CHUNK)], s3)
            h1.wait()
            h2.wait()
            h3.wait()

            for g in range(-(-CHUNK // LANES)):
                nv = min(LANES, CHUNK - g * LANES)
                rowids = g * LANES + lanes

                def j_body(j, acc):
                    jv = jnp.full((LANES,), j, dtype=jnp.int32)
                    vs = plsc.load_gather(srows, [rowids, jv])
                    vd = plsc.load_gather(drows, [rowids, jv])
                    vr = plsc.load_gather(rrows, [rowids, jv])
                    t = (vs + vd) - vr
                    return acc + t * t

                acc = lax.fori_loop(
                    0, D, j_body, jnp.zeros((LANES,), jnp.float32),
                    unroll=8)
                if nv == LANES:
                    out_v[pl.ds(off + g * LANES, LANES)] = -acc
                else:
                    plsc.store_scatter(
                        out_v, [off + g * LANES + lanes], -acc,
                        mask=lanes < nv)
            return carry

        lax.fori_loop(0, nchunks, chunk_body, 0)
        pltpu.sync_copy(out_v, out_hbm.at[pl.ds(base, epw)])

    return scorer


def kernel(z, edge_index, edge_type, rel_emb):
    zn = _normalize(z)
    src = edge_index[0].astype(jnp.int32)
    dst = edge_index[1].astype(jnp.int32)
    rt = edge_type.astype(jnp.int32)
    scorer = _make_sc_scorer(edge_type.shape[0])
    return scorer(zn, src, dst, rt, rel_emb)


# speedup vs baseline: 1.1125x; 1.1125x over previous
import functools

import jax
import jax.numpy as jnp
from jax import lax
from jax.experimental import pallas as pl
from jax.experimental.pallas import tpu as pltpu
from jax.experimental.pallas import tpu_sc as plsc

D = 256
LANES = 16
CHUNK = 40


def _norm_body(z_ref, o_ref):
    x = z_ref[...]
    ss = jnp.sum(x * x, axis=1, keepdims=True)
    nrm = jnp.sqrt(ss)
    o_ref[...] = x / jnp.maximum(nrm, 1e-12)


def _normalize(z):
    n, d = z.shape
    blk = 1000
    return pl.pallas_call(
        _norm_body,
        out_shape=jax.ShapeDtypeStruct((n, d), jnp.float32),
        grid=(n // blk,),
        in_specs=[pl.BlockSpec((blk, d), lambda i: (i, 0))],
        out_specs=pl.BlockSpec((blk, d), lambda i: (i, 0)),
    )(z)


@functools.cache
def _make_sc_scorer(num_edges):
    info = plsc.get_sparse_core_info()
    nc, ns = info.num_cores, info.num_subcores
    nw = nc * ns
    epw = num_edges // nw
    assert epw * nw == num_edges and epw % CHUNK == 0
    nchunks = epw // CHUNK

    mesh = plsc.VectorSubcoreMesh(core_axis_name="c", subcore_axis_name="s")

    @functools.partial(
        pl.kernel,
        mesh=mesh,
        out_type=jax.ShapeDtypeStruct((num_edges,), jnp.float32),
        compiler_params=pltpu.CompilerParams(
            use_tc_tiling_on_sc=False, needs_layout_passes=False),
        scratch_types=[
            pltpu.VMEM((epw,), jnp.int32),
            pltpu.VMEM((epw,), jnp.int32),
            pltpu.VMEM((epw,), jnp.int32),
            pltpu.VMEM((epw,), jnp.float32),
            pltpu.VMEM((CHUNK + 8, D), jnp.float32),
            pltpu.VMEM((CHUNK + 8, D), jnp.float32),
            pltpu.VMEM((CHUNK + 8, D), jnp.float32),
            pltpu.VMEM((CHUNK + 8, D), jnp.float32),
            pltpu.VMEM((CHUNK + 8, D), jnp.float32),
            pltpu.VMEM((CHUNK + 8, D), jnp.float32),
            pltpu.SemaphoreType.DMA,
            pltpu.SemaphoreType.DMA,
        ],
    )
    def scorer(zn_hbm, src_hbm, dst_hbm, rt_hbm, rel_hbm, out_hbm,
               src_v, dst_v, rt_v, out_v,
               srows0, drows0, rrows0, srows1, drows1, rrows1, s0, s1):
        wid = lax.axis_index("s") * nc + lax.axis_index("c")
        base = pl.multiple_of(wid * epw, 8)
        pltpu.sync_copy(src_hbm.at[pl.ds(base, epw)], src_v)
        pltpu.sync_copy(dst_hbm.at[pl.ds(base, epw)], dst_v)
        pltpu.sync_copy(rt_hbm.at[pl.ds(base, epw)], rt_v)

        lanes = lax.iota(jnp.int32, LANES)
        bufs = ((srows0, drows0, rrows0, s0), (srows1, drows1, rrows1, s1))

        def _descs(c, b):
            off = pl.multiple_of(c * CHUNK, 8)
            sr, dr, rr, sem = bufs[b]
            return (
                pltpu.make_async_copy(
                    zn_hbm.at[src_v.at[pl.ds(off, CHUNK)]],
                    sr.at[pl.ds(0, CHUNK)], sem),
                pltpu.make_async_copy(
                    zn_hbm.at[dst_v.at[pl.ds(off, CHUNK)]],
                    dr.at[pl.ds(0, CHUNK)], sem),
                pltpu.make_async_copy(
                    rel_hbm.at[rt_v.at[pl.ds(off, CHUNK)]],
                    rr.at[pl.ds(0, CHUNK)], sem),
            )

        def _fire(c, b):
            for h in _descs(c, b):
                h.start()

        def _wait(c, b):
            for h in _descs(c, b):
                h.wait()

        def _compute(c, b):
            off = pl.multiple_of(c * CHUNK, 8)
            sr, dr, rr, _ = bufs[b]
            for g in range(-(-CHUNK // LANES)):
                nv = min(LANES, CHUNK - g * LANES)
                rowids = g * LANES + lanes

                def j_body(j, acc):
                    jv = jnp.full((LANES,), j, dtype=jnp.int32)
                    vs = plsc.load_gather(sr, [rowids, jv])
                    vd = plsc.load_gather(dr, [rowids, jv])
                    vr = plsc.load_gather(rr, [rowids, jv])
                    t = (vs + vd) - vr
                    return acc + t * t

                acc = lax.fori_loop(
                    0, D, j_body, jnp.zeros((LANES,), jnp.float32),
                    unroll=8)
                if nv == LANES:
                    out_v[pl.ds(off + g * LANES, LANES)] = -acc
                else:
                    plsc.store_scatter(
                        out_v, [off + g * LANES + lanes], -acc,
                        mask=lanes < nv)

        _fire(0, 0)

        def pair_body(i, carry):
            c = pl.multiple_of(i * 2, 2)
            _fire(c + 1, 1)
            _wait(c, 0)
            _compute(c, 0)
            _fire(c + 2, 0)
            _wait(c + 1, 1)
            _compute(c + 1, 1)
            return carry

        lax.fori_loop(0, (nchunks - 1) // 2, pair_body, 0)
        _wait(nchunks - 1, 0)
        _compute(nchunks - 1, 0)
        pltpu.sync_copy(out_v, out_hbm.at[pl.ds(base, epw)])

    return scorer


def kernel(z, edge_index, edge_type, rel_emb):
    zn = _normalize(z)
    src = edge_index[0].astype(jnp.int32)
    dst = edge_index[1].astype(jnp.int32)
    rt = edge_type.astype(jnp.int32)
    scorer = _make_sc_scorer(edge_type.shape[0])
    return scorer(zn, src, dst, rt, rel_emb)


# speedup vs baseline: 8.3157x; 7.4751x over previous
import functools

import jax
import jax.numpy as jnp
from jax import lax
from jax.experimental import pallas as pl
from jax.experimental.pallas import tpu as pltpu
from jax.experimental.pallas import tpu_sc as plsc

D = 256
LANES = 16
CHUNK = 40


def _norm_body(z_ref, o_ref):
    x = z_ref[...]
    ss = jnp.sum(x * x, axis=1, keepdims=True)
    nrm = jnp.sqrt(ss)
    o_ref[...] = x / jnp.maximum(nrm, 1e-12)


def _normalize(z):
    n, d = z.shape
    blk = 1000
    return pl.pallas_call(
        _norm_body,
        out_shape=jax.ShapeDtypeStruct((n, d), jnp.float32),
        grid=(n // blk,),
        in_specs=[pl.BlockSpec((blk, d), lambda i: (i, 0))],
        out_specs=pl.BlockSpec((blk, d), lambda i: (i, 0)),
    )(z)


@functools.cache
def _make_sc_scorer(num_edges):
    info = plsc.get_sparse_core_info()
    nc, ns = info.num_cores, info.num_subcores
    nw = nc * ns
    epw = num_edges // nw
    assert epw * nw == num_edges and epw % CHUNK == 0
    nchunks = epw // CHUNK

    mesh = plsc.VectorSubcoreMesh(core_axis_name="c", subcore_axis_name="s")

    @functools.partial(
        pl.kernel,
        mesh=mesh,
        out_type=jax.ShapeDtypeStruct((num_edges,), jnp.float32),
        compiler_params=pltpu.CompilerParams(
            use_tc_tiling_on_sc=False, needs_layout_passes=False),
        scratch_types=[
            pltpu.VMEM((epw,), jnp.int32),
            pltpu.VMEM((epw,), jnp.int32),
            pltpu.VMEM((epw,), jnp.int32),
            pltpu.VMEM((epw,), jnp.float32),
            pltpu.VMEM((CHUNK + 8, D), jnp.float32),
            pltpu.VMEM((CHUNK + 8, D), jnp.float32),
            pltpu.VMEM((CHUNK + 8, D), jnp.float32),
            pltpu.VMEM((CHUNK + 8, D), jnp.float32),
            pltpu.VMEM((CHUNK + 8, D), jnp.float32),
            pltpu.VMEM((CHUNK + 8, D), jnp.float32),
            pltpu.SemaphoreType.DMA,
            pltpu.SemaphoreType.DMA,
        ],
    )
    def scorer(zn_hbm, src_hbm, dst_hbm, rt_hbm, rel_hbm, out_hbm,
               src_v, dst_v, rt_v, out_v,
               srows0, drows0, rrows0, srows1, drows1, rrows1, s0, s1):
        wid = lax.axis_index("s") * nc + lax.axis_index("c")
        base = pl.multiple_of(wid * epw, 8)
        pltpu.sync_copy(src_hbm.at[pl.ds(base, epw)], src_v)
        pltpu.sync_copy(dst_hbm.at[pl.ds(base, epw)], dst_v)
        pltpu.sync_copy(rt_hbm.at[pl.ds(base, epw)], rt_v)

        lanes = lax.iota(jnp.int32, LANES)
        bufs = ((srows0, drows0, rrows0, s0), (srows1, drows1, rrows1, s1))

        def _descs(c, b):
            off = pl.multiple_of(c * CHUNK, 8)
            sr, dr, rr, sem = bufs[b]
            return (
                pltpu.make_async_copy(
                    zn_hbm.at[src_v.at[pl.ds(off, CHUNK)]],
                    sr.at[pl.ds(0, CHUNK)], sem),
                pltpu.make_async_copy(
                    zn_hbm.at[dst_v.at[pl.ds(off, CHUNK)]],
                    dr.at[pl.ds(0, CHUNK)], sem),
                pltpu.make_async_copy(
                    rel_hbm.at[rt_v.at[pl.ds(off, CHUNK)]],
                    rr.at[pl.ds(0, CHUNK)], sem),
            )

        def _fire(c, b):
            for h in _descs(c, b):
                h.start()

        def _wait(c, b):
            for h in _descs(c, b):
                h.wait()

        def _compute(c, b):
            off = pl.multiple_of(c * CHUNK, 8)
            sr, dr, rr, _ = bufs[b]
            for g in range(-(-CHUNK // LANES)):
                nv = min(LANES, CHUNK - g * LANES)
                rowids = g * LANES + lanes

                def j_body(j, acc):
                    jv = (jnp.full((LANES,), j, dtype=jnp.int32) + lanes) & (D - 1)
                    vs = plsc.load_gather(sr, [rowids, jv])
                    vd = plsc.load_gather(dr, [rowids, jv])
                    vr = plsc.load_gather(rr, [rowids, jv])
                    t = (vs + vd) - vr
                    return acc + t * t

                acc = lax.fori_loop(
                    0, D, j_body, jnp.zeros((LANES,), jnp.float32),
                    unroll=8)
                if nv == LANES:
                    out_v[pl.ds(off + g * LANES, LANES)] = -acc
                else:
                    plsc.store_scatter(
                        out_v, [off + g * LANES + lanes], -acc,
                        mask=lanes < nv)

        _fire(0, 0)

        def pair_body(i, carry):
            c = pl.multiple_of(i * 2, 2)
            _fire(c + 1, 1)
            _wait(c, 0)
            _compute(c, 0)
            _fire(c + 2, 0)
            _wait(c + 1, 1)
            _compute(c + 1, 1)
            return carry

        lax.fori_loop(0, (nchunks - 1) // 2, pair_body, 0)
        _wait(nchunks - 1, 0)
        _compute(nchunks - 1, 0)
        pltpu.sync_copy(out_v, out_hbm.at[pl.ds(base, epw)])

    return scorer


def kernel(z, edge_index, edge_type, rel_emb):
    zn = _normalize(z)
    src = edge_index[0].astype(jnp.int32)
    dst = edge_index[1].astype(jnp.int32)
    rt = edge_type.astype(jnp.int32)
    scorer = _make_sc_scorer(edge_type.shape[0])
    return scorer(zn, src, dst, rt, rel_emb)
